# baseline (speedup 1.0000x reference)
import functools

import jax
import jax.numpy as jnp
from jax import lax
from jax.experimental import pallas as pl
from jax.experimental.pallas import tpu as pltpu
from jax.experimental.pallas import tpu_sc as plsc


def _make_sc_kernel(n_words, words_per_seq, NC, NS):
    NW = NC * NS
    wpw = n_words // NW
    CW = 32 * 1024
    n_chunks = wpw // CW
    w_per_seq = words_per_seq // wpw

    mesh = plsc.VectorSubcoreMesh(core_axis_name="c", subcore_axis_name="s")

    @functools.partial(
        pl.kernel,
        out_type=jax.ShapeDtypeStruct((n_words,), jnp.float32),
        mesh=mesh,
        scratch_types=[
            pltpu.VMEM((CW,), jnp.float32),
            pltpu.VMEM((CW,), jnp.float32),
        ],
    )
    def body(x_hbm, pos_hbm, out_hbm, xbuf, pbuf):
        w = lax.axis_index("s") * NC + lax.axis_index("c")
        base = w * wpw
        pbase = lax.rem(w, w_per_seq) * wpw
        for j in range(n_chunks):
            off = base + j * CW
            poff = pbase + j * CW
            pltpu.sync_copy(x_hbm.at[pl.ds(off, CW)], xbuf)
            pltpu.sync_copy(pos_hbm.at[pl.ds(poff, CW)], pbuf)

            @plsc.parallel_loop(0, CW, step=16, unroll=8)
            def _(i):
                xbuf[pl.ds(i, 16)] = xbuf[pl.ds(i, 16)] + pbuf[pl.ds(i, 16)]

            pltpu.sync_copy(xbuf, out_hbm.at[pl.ds(off, CW)])

    return body


def kernel(x, pos_emb):
    B, T, C = x.shape
    info = plsc.get_sparse_core_info()
    xf = x.reshape(B * T * C)
    pf = pos_emb.reshape(T * C)
    fn = _make_sc_kernel(B * T * C, T * C, info.num_cores, info.num_subcores)
    out = fn(xf, pf)
    return out.reshape(B, T, C)


# speedup vs baseline: 1.3603x; 1.3603x over previous
import functools

import jax
import jax.numpy as jnp
from jax import lax
from jax.experimental import pallas as pl
from jax.experimental.pallas import tpu as pltpu
from jax.experimental.pallas import tpu_sc as plsc

_NSET = 3
_CROWS = 8


def _make_sc_kernel(B, T, C, NC, NS):
    NW = NC * NS
    pos_rows_per_w = T // NW
    CW = _CROWS * C
    n_chunks = pos_rows_per_w // _CROWS
    seq_words = T * C

    mesh = plsc.VectorSubcoreMesh(core_axis_name="c", subcore_axis_name="s")

    scratch = []
    for _ in range(_NSET):
        scratch.append([pltpu.VMEM((CW,), jnp.float32) for _ in range(B)])
        scratch.append(pltpu.VMEM((CW,), jnp.float32))
        scratch.append(pltpu.SemaphoreType.DMA)
        scratch.append(pltpu.SemaphoreType.DMA)
        scratch.append(pltpu.SemaphoreType.DMA)

    @functools.partial(
        pl.kernel,
        out_type=jax.ShapeDtypeStruct((B * seq_words,), jnp.float32),
        mesh=mesh,
        scratch_types=scratch,
    )
    def body(x_hbm, pos_hbm, out_hbm, *sets):
        w = lax.axis_index("s") * NC + lax.axis_index("c")
        pw0 = w * (pos_rows_per_w * C)

        def chunk_off(c):
            return pw0 + c * CW

        ld_descs = {}
        st_descs = {}

        def issue_loads(c):
            xbufs, pbuf, xsem, psem, _ = sets[5 * (c % _NSET):5 * (c % _NSET) + 5]
            off = chunk_off(c)
            descs = [
                pltpu.async_copy(
                    x_hbm.at[pl.ds(b * seq_words + off, CW)], xbufs[b], xsem
                )
                for b in range(B)
            ]
            descs.append(pltpu.async_copy(pos_hbm.at[pl.ds(off, CW)], pbuf, psem))
            ld_descs[c] = descs

        for c in range(min(_NSET - 1, n_chunks)):
            issue_loads(c)

        for c in range(n_chunks):
            s = c % _NSET
            xbufs, pbuf, _, _, stsem = sets[5 * s:5 * s + 5]
            for d in ld_descs.pop(c):
                d.wait()

            a0, a1, a2, a3 = xbufs

            @plsc.parallel_loop(0, CW, step=16, unroll=2)
            def _(i):
                ds = pl.ds(i, 16)
                pv = pbuf[ds]
                a0[ds] = a0[ds] + pv
                a1[ds] = a1[ds] + pv
                a2[ds] = a2[ds] + pv
                a3[ds] = a3[ds] + pv

            off = chunk_off(c)
            st_descs[c] = [
                pltpu.async_copy(
                    xbufs[b], out_hbm.at[pl.ds(b * seq_words + off, CW)], stsem
                )
                for b in range(B)
            ]

            nxt = c + _NSET - 1
            if nxt < n_chunks:
                for d in st_descs.pop(c - 1, ()):
                    d.wait()
                issue_loads(nxt)

        for c in sorted(st_descs):
            for d in st_descs[c]:
                d.wait()

    return body


def kernel(x, pos_emb):
    B, T, C = x.shape
    info = plsc.get_sparse_core_info()
    fn = _make_sc_kernel(B, T, C, info.num_cores, info.num_subcores)
    out = fn(x.reshape(-1), pos_emb.reshape(-1))
    return out.reshape(B, T, C)


# speedup vs baseline: 4.0903x; 3.0070x over previous
import functools

import jax
import jax.numpy as jnp
from jax import lax
from jax.experimental import pallas as pl
from jax.experimental.pallas import tpu as pltpu
from jax.experimental.pallas import tpu_sc as plsc

_NSET = 3
_CROWS = 8


def _make_sc_kernel(B, T, C, NC, NS):
    NW = NC * NS
    pos_rows_per_w = T // NW
    n_chunks = pos_rows_per_w // _CROWS

    mesh = plsc.VectorSubcoreMesh(core_axis_name="c", subcore_axis_name="s")

    scratch = []
    for _ in range(_NSET):
        scratch.append([pltpu.VMEM((_CROWS, C), jnp.float32) for _ in range(B)])
        scratch.append(pltpu.VMEM((_CROWS, C), jnp.float32))
        scratch.append(pltpu.SemaphoreType.DMA)
        scratch.append(pltpu.SemaphoreType.DMA)
        scratch.append(pltpu.SemaphoreType.DMA)

    @functools.partial(
        pl.kernel,
        out_type=jax.ShapeDtypeStruct((B, T, C), jnp.float32),
        mesh=mesh,
        scratch_types=scratch,
    )
    def body(x_3d, pos_hbm, out_3d, *sets):
        x_hbm = x_3d.reshape(B * T, C)
        out_hbm = out_3d.reshape(B * T, C)
        w = lax.axis_index("s") * NC + lax.axis_index("c")
        pr0 = w * pos_rows_per_w

        ld_descs = {}
        st_descs = {}

        def issue_loads(c):
            xbufs, pbuf, xsem, psem, _ = sets[5 * (c % _NSET):5 * (c % _NSET) + 5]
            row = pr0 + c * _CROWS
            descs = [
                pltpu.async_copy(
                    x_hbm.at[pl.ds(b * T + row, _CROWS)], xbufs[b], xsem
                )
                for b in range(B)
            ]
            descs.append(
                pltpu.async_copy(pos_hbm.at[pl.ds(row, _CROWS)], pbuf, psem)
            )
            ld_descs[c] = descs

        for c in range(min(_NSET - 1, n_chunks)):
            issue_loads(c)

        for c in range(n_chunks):
            s = c % _NSET
            xbufs, pbuf, _, _, stsem = sets[5 * s:5 * s + 5]
            for d in ld_descs.pop(c):
                d.wait()

            a0, a1, a2, a3 = xbufs

            @plsc.parallel_loop(0, _CROWS * C, step=16, unroll=2)
            def _(i):
                r = lax.shift_right_logical(i, 10)
                ds = pl.ds(pl.multiple_of(lax.bitwise_and(i, C - 1), 16), 16)
                pv = pbuf[r, ds]
                a0[r, ds] = a0[r, ds] + pv
                a1[r, ds] = a1[r, ds] + pv
                a2[r, ds] = a2[r, ds] + pv
                a3[r, ds] = a3[r, ds] + pv

            row = pr0 + c * _CROWS
            st_descs[c] = [
                pltpu.async_copy(
                    xbufs[b], out_hbm.at[pl.ds(b * T + row, _CROWS)], stsem
                )
                for b in range(B)
            ]

            nxt = c + _NSET - 1
            if nxt < n_chunks:
                for d in st_descs.pop(c - 1, ()):
                    d.wait()
                issue_loads(nxt)

        for c in sorted(st_descs):
            for d in st_descs[c]:
                d.wait()

    return body


def kernel(x, pos_emb):
    B, T, C = x.shape
    info = plsc.get_sparse_core_info()
    fn = _make_sc_kernel(B, T, C, info.num_cores, info.num_subcores)
    return fn(x, pos_emb)
